# baseline (speedup 1.0000x reference)
import dataclasses
import functools

import jax
import jax.numpy as jnp
from jax import lax
from jax.experimental import pallas as pl
from jax.experimental.pallas import tpu as pltpu
from jax.experimental.pallas import tpu_sc as plsc

N = 10000
E = 320000
NW = 32
R = 320
NPAD = NW * R
RACC = R + 8
EBLK = 2000
NCHUNK = EBLK // 16
NBLK = E // EBLK
GB = 64
ROW_BLK = 2000


def _segmax_sc(table, ei_flat, F):
    mesh = plsc.VectorSubcoreMesh(core_axis_name="c", subcore_axis_name="s")
    cp = pltpu.CompilerParams()
    if "needs_layout_passes" in pltpu.CompilerParams.__dataclass_fields__:
        cp = dataclasses.replace(cp, needs_layout_passes=False)

    @functools.partial(
        pl.kernel,
        out_type=jax.ShapeDtypeStruct((NPAD * F,), jnp.float32),
        mesh=mesh,
        compiler_params=cp,
        scratch_types=[
            pltpu.VMEM((RACC * F,), jnp.float32),
            pltpu.VMEM((EBLK + GB,), jnp.int32),
            pltpu.VMEM((EBLK + GB,), jnp.int32),
            pltpu.VMEM((EBLK,), jnp.int32),
            pltpu.VMEM((EBLK,), jnp.int32),
            pltpu.VMEM((GB, F), jnp.float32),
            pltpu.VMEM((32,), jnp.int32),
        ],
    )
    def k(table_hbm, ei_hbm, out_hbm, acc, psrc, pdst, sbuf, dbuf, gbuf, rstage):
        wid = lax.axis_index("s") * 2 + lax.axis_index("c")
        lo = wid * R
        neg = jnp.full((16,), -jnp.inf, jnp.float32)
        iota16 = lax.iota(jnp.int32, 16)

        @pl.loop(0, RACC * F // 16)
        def _(r):
            acc[pl.ds(r * 16, 16)] = neg

        @pl.loop(0, NBLK)
        def _(blk):
            pltpu.sync_copy(ei_hbm.at[pl.ds(blk * EBLK, EBLK)], sbuf)
            pltpu.sync_copy(ei_hbm.at[pl.ds(E + blk * EBLK, EBLK)], dbuf)

            def chunk(j, offv):
                dv = dbuf[pl.ds(j * 16, 16)]
                sv = sbuf[pl.ds(j * 16, 16)]
                m = (dv >= lo) & (dv < lo + R)
                pos = offv + plsc.cumsum(m.astype(jnp.int32)) - 1
                pos = jnp.where(m, pos, 0)
                plsc.store_scatter(pdst, [pos], dv - lo, mask=m)
                plsc.store_scatter(psrc, [pos], sv, mask=m)
                return offv + plsc.all_reduce_population_count(m)

            offv = lax.fori_loop(
                0, NCHUNK, chunk, jnp.zeros((16,), jnp.int32))
            p = offv[0]

            for t in range(GB // 16):
                pdst[pl.ds(p + t * 16, 16)] = jnp.full((16,), RACC - 1, jnp.int32)
                psrc[pl.ds(p + t * 16, 16)] = jnp.zeros((16,), jnp.int32)
            nsub = (p + GB - 1) >> 6

            def sub(b, carry):
                pltpu.sync_copy(table_hbm.at[psrc.at[pl.ds(b * GB, GB)]], gbuf)

                @pl.loop(0, GB // 16)
                def _(q):
                    dv = pdst[pl.ds(b * GB + q * 16, 16)]
                    rstage[pl.ds(16, 16)] = dv * F
                    for l in range(16):
                        roff = plsc.load_gather(
                            rstage, [jnp.full((16,), 16 + l, jnp.int32)])
                        for c in range(F // 16):
                            idx = roff + (c * 16 + iota16)
                            old = plsc.load_gather(acc, [idx])
                            new = jnp.maximum(
                                old, gbuf[q * 16 + l, pl.ds(c * 16, 16)])
                            plsc.store_scatter(acc, [idx], new)

                return carry

            lax.fori_loop(0, nsub, sub, jnp.int32(0))

        pltpu.sync_copy(
            acc.at[pl.ds(0, R * F)], out_hbm.at[pl.ds(wid * R * F, R * F)])

    return k(table, ei_flat).reshape(NPAD, F)


def _layer_body(aggr_ref, h_ref, wlT_ref, wrT_ref, bl_ref, o_ref, *, final):
    a = aggr_ref[...]
    a = jnp.where(jnp.isfinite(a), a, 0.0)
    acc = jnp.dot(a, wlT_ref[...], preferred_element_type=jnp.float32)
    acc += jnp.dot(h_ref[...], wrT_ref[...], preferred_element_type=jnp.float32)
    acc += bl_ref[...]
    if final:
        m = jnp.max(acc, axis=1, keepdims=True)
        z = acc - m
        lse = jnp.log(jnp.sum(jnp.exp(z), axis=1, keepdims=True))
        acc = z - lse
    o_ref[...] = acc


def _tc_layer(aggr, h, wlT, wrT, bl, *, final=False):
    fin = h.shape[1]
    hout = wlT.shape[1]
    return pl.pallas_call(
        functools.partial(_layer_body, final=final),
        grid=(N // ROW_BLK,),
        in_specs=[
            pl.BlockSpec((ROW_BLK, fin), lambda i: (i, 0)),
            pl.BlockSpec((ROW_BLK, fin), lambda i: (i, 0)),
            pl.BlockSpec((fin, hout), lambda i: (0, 0)),
            pl.BlockSpec((fin, hout), lambda i: (0, 0)),
            pl.BlockSpec((1, hout), lambda i: (0, 0)),
        ],
        out_specs=pl.BlockSpec((ROW_BLK, hout), lambda i: (i, 0)),
        out_shape=jax.ShapeDtypeStruct((N, hout), jnp.float32),
    )(aggr, h, wlT, wrT, bl)


def _pad2(a, r, c):
    return jnp.zeros((r, c), a.dtype).at[: a.shape[0], : a.shape[1]].set(a)


def kernel(x, edge_index, Wl1, bl1, Wr1, Wl2, bl2, Wr2, Wl3, bl3, Wr3):
    ei_flat = edge_index.reshape(2 * E)

    wlT1 = _pad2(Wl1.T, 128, 256)
    wrT1 = _pad2(Wr1.T, 128, 256)
    b1 = _pad2(bl1[None, :], 1, 256)
    wlT2 = _pad2(Wl2.T, 256, 128)
    wrT2 = _pad2(Wr2.T, 256, 128)
    b2 = _pad2(bl2[None, :], 1, 128)
    wlT3 = _pad2(Wl3.T, 128, 16)
    wrT3 = _pad2(Wr3.T, 128, 16)
    b3 = _pad2(bl3[None, :], 1, 16)

    aggr1 = _segmax_sc(x, ei_flat, 128)
    h1 = _tc_layer(aggr1, x, wlT1, wrT1, b1)

    aggr2 = _segmax_sc(h1, ei_flat, 256)
    h2 = _tc_layer(aggr2, h1, wlT2, wrT2, b2)

    aggr3 = _segmax_sc(h2, ei_flat, 128)
    out = _tc_layer(aggr3, h2, wlT3, wrT3, b3, final=True)
    return out


# speedup vs baseline: 1.9194x; 1.9194x over previous
import dataclasses
import functools

import jax
import jax.numpy as jnp
from jax import lax
from jax.experimental import pallas as pl
from jax.experimental.pallas import tpu as pltpu
from jax.experimental.pallas import tpu_sc as plsc

N = 10000
E = 320000
NW = 32
R = 320
NPAD = NW * R
RACC = R + 8
ROW_BLK = 2000


def _segmax_sc(table, ei_flat, F, EBLK, GB):
    NCHUNK = EBLK // 16
    NBLK = E // EBLK
    GSH = GB.bit_length() - 1
    mesh = plsc.VectorSubcoreMesh(core_axis_name="c", subcore_axis_name="s")
    cp = pltpu.CompilerParams()
    if "needs_layout_passes" in pltpu.CompilerParams.__dataclass_fields__:
        cp = dataclasses.replace(cp, needs_layout_passes=False)

    @functools.partial(
        pl.kernel,
        out_type=jax.ShapeDtypeStruct((NPAD * F,), jnp.float32),
        mesh=mesh,
        compiler_params=cp,
        scratch_types=[
            pltpu.VMEM((RACC * F,), jnp.float32),
            pltpu.VMEM((EBLK + GB,), jnp.int32),
            pltpu.VMEM((EBLK + GB,), jnp.int32),
            pltpu.VMEM((EBLK,), jnp.int32),
            pltpu.VMEM((EBLK,), jnp.int32),
            pltpu.VMEM((GB, F), jnp.float32),
            pltpu.VMEM((32,), jnp.int32),
        ],
    )
    def k(table_hbm, ei_hbm, out_hbm, acc, psrc, pdst, sbuf, dbuf, gbuf, rstage):
        wid = lax.axis_index("s") * 2 + lax.axis_index("c")
        lo = wid * R
        neg = jnp.full((16,), -jnp.inf, jnp.float32)
        iota16 = lax.iota(jnp.int32, 16)

        @pl.loop(0, RACC * F // 16)
        def _(r):
            acc[pl.ds(r * 16, 16)] = neg

        @pl.loop(0, NBLK)
        def _(blk):
            pltpu.sync_copy(ei_hbm.at[pl.ds(blk * EBLK, EBLK)], sbuf)
            pltpu.sync_copy(ei_hbm.at[pl.ds(E + blk * EBLK, EBLK)], dbuf)

            def chunk(j, offv):
                dv = dbuf[pl.ds(j * 16, 16)]
                sv = sbuf[pl.ds(j * 16, 16)]
                m = (dv >= lo) & (dv < lo + R)
                pos = offv + plsc.cumsum(m.astype(jnp.int32)) - 1
                pos = jnp.where(m, pos, 0)
                plsc.store_scatter(pdst, [pos], dv - lo, mask=m)
                plsc.store_scatter(psrc, [pos], sv, mask=m)
                return offv + plsc.all_reduce_population_count(m)

            offv = lax.fori_loop(
                0, NCHUNK, chunk, jnp.zeros((16,), jnp.int32))
            p = offv[0]

            for t in range(GB // 16):
                pdst[pl.ds(p + t * 16, 16)] = jnp.full((16,), RACC - 1, jnp.int32)
                psrc[pl.ds(p + t * 16, 16)] = jnp.zeros((16,), jnp.int32)
            nsub = (p + GB - 1) >> GSH

            def sub(b, carry):
                pltpu.sync_copy(table_hbm.at[psrc.at[pl.ds(b * GB, GB)]], gbuf)

                @pl.loop(0, GB // 16)
                def _(q):
                    dv = pdst[pl.ds(b * GB + q * 16, 16)]
                    rstage[pl.ds(16, 16)] = dv * F
                    for l in range(16):
                        roff = plsc.load_gather(
                            rstage, [jnp.full((16,), 16 + l, jnp.int32)])
                        for c in range(F // 16):
                            idx = roff + (c * 16 + iota16)
                            old = plsc.load_gather(acc, [idx])
                            new = jnp.maximum(
                                old, gbuf[q * 16 + l, pl.ds(c * 16, 16)])
                            plsc.store_scatter(acc, [idx], new)

                return carry

            lax.fori_loop(0, nsub, sub, jnp.int32(0))

        pltpu.sync_copy(
            acc.at[pl.ds(0, R * F)], out_hbm.at[pl.ds(wid * R * F, R * F)])

    return k(table, ei_flat).reshape(NPAD, F)


def _layer_body(aggr_ref, h_ref, wlT_ref, wrT_ref, bl_ref, o_ref, *, final):
    a = aggr_ref[...]
    a = jnp.where(jnp.isfinite(a), a, 0.0)
    acc = jnp.dot(a, wlT_ref[...], preferred_element_type=jnp.float32)
    acc += jnp.dot(h_ref[...], wrT_ref[...], preferred_element_type=jnp.float32)
    acc += bl_ref[...]
    if final:
        m = jnp.max(acc, axis=1, keepdims=True)
        z = acc - m
        lse = jnp.log(jnp.sum(jnp.exp(z), axis=1, keepdims=True))
        acc = z - lse
    o_ref[...] = acc


def _tc_layer(aggr, h, wlT, wrT, bl, *, final=False):
    fin = h.shape[1]
    hout = wlT.shape[1]
    return pl.pallas_call(
        functools.partial(_layer_body, final=final),
        grid=(N // ROW_BLK,),
        in_specs=[
            pl.BlockSpec((ROW_BLK, fin), lambda i: (i, 0)),
            pl.BlockSpec((ROW_BLK, fin), lambda i: (i, 0)),
            pl.BlockSpec((fin, hout), lambda i: (0, 0)),
            pl.BlockSpec((fin, hout), lambda i: (0, 0)),
            pl.BlockSpec((1, hout), lambda i: (0, 0)),
        ],
        out_specs=pl.BlockSpec((ROW_BLK, hout), lambda i: (i, 0)),
        out_shape=jax.ShapeDtypeStruct((N, hout), jnp.float32),
    )(aggr, h, wlT, wrT, bl)


def _pad2(a, r, c):
    return jnp.zeros((r, c), a.dtype).at[: a.shape[0], : a.shape[1]].set(a)


def kernel(x, edge_index, Wl1, bl1, Wr1, Wl2, bl2, Wr2, Wl3, bl3, Wr3):
    ei_flat = edge_index.reshape(2 * E)

    wlT1 = _pad2(Wl1.T, 128, 256)
    wrT1 = _pad2(Wr1.T, 128, 256)
    b1 = _pad2(bl1[None, :], 1, 256)
    wlT2 = _pad2(Wl2.T, 256, 128)
    wrT2 = _pad2(Wr2.T, 256, 128)
    b2 = _pad2(bl2[None, :], 1, 128)
    wlT3 = _pad2(Wl3.T, 128, 16)
    wrT3 = _pad2(Wr3.T, 128, 16)
    b3 = _pad2(bl3[None, :], 1, 16)

    aggr1 = _segmax_sc(x, ei_flat, 128, 8000, 128)
    h1 = _tc_layer(aggr1, x, wlT1, wrT1, b1)

    aggr2 = _segmax_sc(h1, ei_flat, 256, 4000, 64)
    h2 = _tc_layer(aggr2, h1, wlT2, wrT2, b2)

    aggr3 = _segmax_sc(h2, ei_flat, 128, 8000, 128)
    out = _tc_layer(aggr3, h2, wlT3, wrT3, b3, final=True)
    return out


# speedup vs baseline: 3.8106x; 1.9853x over previous
import dataclasses
import functools

import jax
import jax.numpy as jnp
from jax import lax
from jax.experimental import pallas as pl
from jax.experimental.pallas import tpu as pltpu
from jax.experimental.pallas import tpu_sc as plsc

N = 10000
E = 320000
NW = 32
R = 320
NPAD = NW * R
RACC = R + 4
ROW_BLK = 2000


def _segmax_sc(table, ei_flat, F, EBLK, GB):
    NCHUNK = EBLK // 16
    NBLK = E // EBLK
    GSH = GB.bit_length() - 1
    PCAP = EBLK + GB
    mesh = plsc.VectorSubcoreMesh(core_axis_name="c", subcore_axis_name="s")
    cp = pltpu.CompilerParams()
    if "needs_layout_passes" in pltpu.CompilerParams.__dataclass_fields__:
        cp = dataclasses.replace(cp, needs_layout_passes=False)

    @functools.partial(
        pl.kernel,
        out_type=jax.ShapeDtypeStruct((NPAD * F,), jnp.float32),
        mesh=mesh,
        compiler_params=cp,
        scratch_types=[
            pltpu.VMEM((RACC * F,), jnp.float32),
            pltpu.VMEM((PCAP,), jnp.int32),
            pltpu.VMEM((PCAP,), jnp.int32),
            pltpu.VMEM((EBLK,), jnp.int32),
            pltpu.VMEM((EBLK,), jnp.int32),
            pltpu.VMEM((EBLK,), jnp.int32),
            pltpu.VMEM((EBLK,), jnp.int32),
            pltpu.VMEM((GB, F), jnp.float32),
            pltpu.VMEM((GB, F), jnp.float32),
            pltpu.VMEM((32,), jnp.int32),
            pltpu.SemaphoreType.DMA,
            pltpu.SemaphoreType.DMA,
            pltpu.SemaphoreType.DMA,
            pltpu.SemaphoreType.DMA,
        ],
    )
    def k(table_hbm, ei_hbm, out_hbm, acc, psrc, pdst,
          sbuf0, dbuf0, sbuf1, dbuf1, gbuf0, gbuf1, rstage,
          semA, semB, semG0, semG1):
        wid = lax.axis_index("s") * 2 + lax.axis_index("c")
        lo = wid * R
        neg = jnp.full((16,), -jnp.inf, jnp.float32)
        iota16 = lax.iota(jnp.int32, 16)
        garb = jnp.full((16,), RACC - 1, jnp.int32)

        @pl.loop(0, RACC * F // 16)
        def _(r):
            acc[pl.ds(r * 16, 16)] = neg

        @pl.loop(0, PCAP // 16)
        def _(i):
            psrc[pl.ds(i * 16, 16)] = jnp.full((16,), wid * R, jnp.int32)
            pdst[pl.ds(i * 16, 16)] = garb

        def start_blk(b, sb, db, sem):
            pltpu.async_copy(ei_hbm.at[pl.ds(b * EBLK, EBLK)], sb, sem)
            pltpu.async_copy(ei_hbm.at[pl.ds(E + b * EBLK, EBLK)], db, sem)

        def wait_blk(sb, db, sem):
            pltpu.make_async_copy(ei_hbm.at[pl.ds(0, EBLK)], sb, sem).wait()
            pltpu.make_async_copy(ei_hbm.at[pl.ds(0, EBLK)], db, sem).wait()

        def scan(sb, db):
            def chunk(j, offv):
                dv = db[pl.ds(j * 16, 16)]
                sv = sb[pl.ds(j * 16, 16)]
                m = (dv >= lo) & (dv < lo + R)
                pos = offv + plsc.cumsum(m.astype(jnp.int32)) - 1
                pos = jnp.where(m, pos, 0)
                plsc.store_scatter(pdst, [pos], dv - lo, mask=m)
                plsc.store_scatter(psrc, [pos], sv, mask=m)
                return offv + plsc.all_reduce_population_count(m)

            return lax.fori_loop(0, NCHUNK, chunk, jnp.zeros((16,), jnp.int32))

        def start_g(b, gb, sem):
            pltpu.async_copy(table_hbm.at[psrc.at[pl.ds(b * GB, GB)]], gb, sem)

        def wait_g(gb, sem):
            pltpu.make_async_copy(
                table_hbm.at[psrc.at[pl.ds(0, GB)]], gb, sem).wait()

        def drain(offv):
            p = offv[0]
            nsub = (p + GB - 1) >> GSH

            def accum(b, gb):
                @pl.loop(0, GB // 16)
                def _(q):
                    base = b * GB + q * 16
                    dv = pdst[pl.ds(base, 16)]
                    dv = jnp.where(base + iota16 < offv, dv, garb)
                    rstage[pl.ds(16, 16)] = dv * F
                    for l in range(16):
                        roff = plsc.load_gather(
                            rstage, [jnp.full((16,), 16 + l, jnp.int32)])
                        for c in range(F // 16):
                            idx = roff + (c * 16 + iota16)
                            old = plsc.load_gather(acc, [idx])
                            new = jnp.maximum(
                                old, gb[q * 16 + l, pl.ds(c * 16, 16)])
                            plsc.store_scatter(acc, [idx], new)

            @pl.when(nsub > 0)
            def _():
                start_g(0, gbuf0, semG0)

            def pair(i, carry):
                b0 = 2 * i
                b1 = b0 + 1

                @pl.when(b1 < nsub)
                def _():
                    start_g(b1, gbuf1, semG1)

                wait_g(gbuf0, semG0)
                accum(b0, gbuf0)

                @pl.when(b0 + 2 < nsub)
                def _():
                    start_g(b0 + 2, gbuf0, semG0)

                @pl.when(b1 < nsub)
                def _():
                    wait_g(gbuf1, semG1)
                    accum(b1, gbuf1)

                return carry

            lax.fori_loop(0, (nsub + 1) >> 1, pair, jnp.int32(0))

        start_blk(0, sbuf0, dbuf0, semA)

        def blkpair(i, carry):
            b0 = 2 * i
            b1 = b0 + 1
            start_blk(b1, sbuf1, dbuf1, semB)
            wait_blk(sbuf0, dbuf0, semA)
            drain(scan(sbuf0, dbuf0))

            @pl.when(b0 + 2 < NBLK)
            def _():
                start_blk(b0 + 2, sbuf0, dbuf0, semA)

            wait_blk(sbuf1, dbuf1, semB)
            drain(scan(sbuf1, dbuf1))
            return carry

        lax.fori_loop(0, NBLK // 2, blkpair, jnp.int32(0))

        pltpu.sync_copy(
            acc.at[pl.ds(0, R * F)], out_hbm.at[pl.ds(wid * R * F, R * F)])

    return k(table, ei_flat).reshape(NPAD, F)


def _layer_body(aggr_ref, h_ref, wlT_ref, wrT_ref, bl_ref, o_ref, *, final):
    a = aggr_ref[...]
    a = jnp.where(jnp.isfinite(a), a, 0.0)
    acc = jnp.dot(a, wlT_ref[...], preferred_element_type=jnp.float32)
    acc += jnp.dot(h_ref[...], wrT_ref[...], preferred_element_type=jnp.float32)
    acc += bl_ref[...]
    if final:
        m = jnp.max(acc, axis=1, keepdims=True)
        z = acc - m
        lse = jnp.log(jnp.sum(jnp.exp(z), axis=1, keepdims=True))
        acc = z - lse
    o_ref[...] = acc


def _tc_layer(aggr, h, wlT, wrT, bl, *, final=False):
    fin = h.shape[1]
    hout = wlT.shape[1]
    return pl.pallas_call(
        functools.partial(_layer_body, final=final),
        grid=(N // ROW_BLK,),
        in_specs=[
            pl.BlockSpec((ROW_BLK, fin), lambda i: (i, 0)),
            pl.BlockSpec((ROW_BLK, fin), lambda i: (i, 0)),
            pl.BlockSpec((fin, hout), lambda i: (0, 0)),
            pl.BlockSpec((fin, hout), lambda i: (0, 0)),
            pl.BlockSpec((1, hout), lambda i: (0, 0)),
        ],
        out_specs=pl.BlockSpec((ROW_BLK, hout), lambda i: (i, 0)),
        out_shape=jax.ShapeDtypeStruct((N, hout), jnp.float32),
    )(aggr, h, wlT, wrT, bl)


def _pad2(a, r, c):
    return jnp.zeros((r, c), a.dtype).at[: a.shape[0], : a.shape[1]].set(a)


def kernel(x, edge_index, Wl1, bl1, Wr1, Wl2, bl2, Wr2, Wl3, bl3, Wr3):
    ei_flat = edge_index.reshape(2 * E)

    wlT1 = _pad2(Wl1.T, 128, 256)
    wrT1 = _pad2(Wr1.T, 128, 256)
    b1 = _pad2(bl1[None, :], 1, 256)
    wlT2 = _pad2(Wl2.T, 256, 128)
    wrT2 = _pad2(Wr2.T, 256, 128)
    b2 = _pad2(bl2[None, :], 1, 128)
    wlT3 = _pad2(Wl3.T, 128, 16)
    wrT3 = _pad2(Wr3.T, 128, 16)
    b3 = _pad2(bl3[None, :], 1, 16)

    aggr1 = _segmax_sc(x, ei_flat, 128, 4000, 128)
    h1 = _tc_layer(aggr1, x, wlT1, wrT1, b1)

    aggr2 = _segmax_sc(h1, ei_flat, 256, 1600, 64)
    h2 = _tc_layer(aggr2, h1, wlT2, wrT2, b2)

    aggr3 = _segmax_sc(h2, ei_flat, 128, 4000, 128)
    out = _tc_layer(aggr3, h2, wlT3, wrT3, b3, final=True)
    return out


# speedup vs baseline: 3.8332x; 1.0059x over previous
import dataclasses
import functools

import jax
import jax.numpy as jnp
from jax import lax
from jax.experimental import pallas as pl
from jax.experimental.pallas import tpu as pltpu
from jax.experimental.pallas import tpu_sc as plsc

N = 10000
E = 320000
NW = 32
R = 320
NPAD = NW * R
RACC = R + 4
ROW_BLK = 2000


def _segmax_sc(table, ei_flat, F, EBLK, GB):
    NCHUNK = EBLK // 16
    NBLK = E // EBLK
    GSH = GB.bit_length() - 1
    PCAP = EBLK + GB
    mesh = plsc.VectorSubcoreMesh(core_axis_name="c", subcore_axis_name="s")
    cp = pltpu.CompilerParams()
    if "needs_layout_passes" in pltpu.CompilerParams.__dataclass_fields__:
        cp = dataclasses.replace(cp, needs_layout_passes=False)

    @functools.partial(
        pl.kernel,
        out_type=jax.ShapeDtypeStruct((NPAD * F,), jnp.float32),
        mesh=mesh,
        compiler_params=cp,
        scratch_types=[
            pltpu.VMEM((RACC * F,), jnp.float32),
            pltpu.VMEM((PCAP,), jnp.int32),
            pltpu.VMEM((PCAP,), jnp.int32),
            pltpu.VMEM((EBLK,), jnp.int32),
            pltpu.VMEM((EBLK,), jnp.int32),
            pltpu.VMEM((EBLK,), jnp.int32),
            pltpu.VMEM((EBLK,), jnp.int32),
            pltpu.VMEM((GB, F), jnp.float32),
            pltpu.VMEM((GB, F), jnp.float32),
            pltpu.VMEM((32,), jnp.int32),
            pltpu.SemaphoreType.DMA,
            pltpu.SemaphoreType.DMA,
            pltpu.SemaphoreType.DMA,
            pltpu.SemaphoreType.DMA,
        ],
    )
    def k(table_hbm, ei_hbm, out_hbm, acc, psrc, pdst,
          sbuf0, dbuf0, sbuf1, dbuf1, gbuf0, gbuf1, rstage,
          semA, semB, semG0, semG1):
        wid = lax.axis_index("s") * 2 + lax.axis_index("c")
        lo = wid * R
        neg = jnp.full((16,), -jnp.inf, jnp.float32)
        iota16 = lax.iota(jnp.int32, 16)
        garb = jnp.full((16,), RACC - 1, jnp.int32)

        @pl.loop(0, RACC * F // 16)
        def _(r):
            acc[pl.ds(r * 16, 16)] = neg

        @pl.loop(0, PCAP // 16)
        def _(i):
            psrc[pl.ds(i * 16, 16)] = jnp.full((16,), wid * R, jnp.int32)
            pdst[pl.ds(i * 16, 16)] = garb

        def start_blk(b, sb, db, sem):
            pltpu.async_copy(ei_hbm.at[pl.ds(b * EBLK, EBLK)], sb, sem)
            pltpu.async_copy(ei_hbm.at[pl.ds(E + b * EBLK, EBLK)], db, sem)

        def wait_blk(sb, db, sem):
            pltpu.make_async_copy(ei_hbm.at[pl.ds(0, EBLK)], sb, sem).wait()
            pltpu.make_async_copy(ei_hbm.at[pl.ds(0, EBLK)], db, sem).wait()

        def scan(sb, db):
            def chunk(j, offv):
                dv = db[pl.ds(j * 16, 16)]
                sv = sb[pl.ds(j * 16, 16)]
                m = (dv >= lo) & (dv < lo + R)
                pos = offv + plsc.cumsum(m.astype(jnp.int32)) - 1
                pos = jnp.where(m, pos, 0)
                plsc.store_scatter(pdst, [pos], dv - lo, mask=m)
                plsc.store_scatter(psrc, [pos], sv, mask=m)
                return offv + plsc.all_reduce_population_count(m)

            return lax.fori_loop(0, NCHUNK, chunk, jnp.zeros((16,), jnp.int32))

        def start_g(b, gb, sem):
            pltpu.async_copy(table_hbm.at[psrc.at[pl.ds(b * GB, GB)]], gb, sem)

        def wait_g(gb, sem):
            pltpu.make_async_copy(
                table_hbm.at[psrc.at[pl.ds(0, GB)]], gb, sem).wait()

        def drain(offv):
            p = offv[0]
            nsub = (p + GB - 1) >> GSH

            def accum(b, gb):
                @pl.loop(0, GB // 16)
                def _(q):
                    base = b * GB + q * 16
                    dv = pdst[pl.ds(base, 16)]
                    dv = jnp.where(base + iota16 < offv, dv, garb)
                    rstage[pl.ds(16, 16)] = dv * F
                    for l in range(16):
                        roff = plsc.load_gather(
                            rstage, [jnp.full((16,), 16 + l, jnp.int32)])
                        for c in range(F // 16):
                            idx = roff + (c * 16 + iota16)
                            old = plsc.load_gather(acc, [idx])
                            new = jnp.maximum(
                                old, gb[q * 16 + l, pl.ds(c * 16, 16)])
                            plsc.store_scatter(acc, [idx], new)

            @pl.when(nsub > 0)
            def _():
                start_g(0, gbuf0, semG0)

            def pair(i, carry):
                b0 = 2 * i
                b1 = b0 + 1

                @pl.when(b1 < nsub)
                def _():
                    start_g(b1, gbuf1, semG1)

                wait_g(gbuf0, semG0)
                accum(b0, gbuf0)

                @pl.when(b0 + 2 < nsub)
                def _():
                    start_g(b0 + 2, gbuf0, semG0)

                @pl.when(b1 < nsub)
                def _():
                    wait_g(gbuf1, semG1)
                    accum(b1, gbuf1)

                return carry

            lax.fori_loop(0, (nsub + 1) >> 1, pair, jnp.int32(0))

        start_blk(0, sbuf0, dbuf0, semA)

        def blkpair(i, carry):
            b0 = 2 * i
            b1 = b0 + 1
            start_blk(b1, sbuf1, dbuf1, semB)
            wait_blk(sbuf0, dbuf0, semA)
            drain(scan(sbuf0, dbuf0))

            @pl.when(b0 + 2 < NBLK)
            def _():
                start_blk(b0 + 2, sbuf0, dbuf0, semA)

            wait_blk(sbuf1, dbuf1, semB)
            drain(scan(sbuf1, dbuf1))
            return carry

        lax.fori_loop(0, NBLK // 2, blkpair, jnp.int32(0))

        pltpu.sync_copy(
            acc.at[pl.ds(0, R * F)], out_hbm.at[pl.ds(wid * R * F, R * F)])

    return k(table, ei_flat).reshape(NPAD, F)


def _layer_body(aggr_ref, h_ref, wlT_ref, wrT_ref, bl_ref, o_ref, *, final):
    a = aggr_ref[...]
    a = jnp.where(jnp.isfinite(a), a, 0.0)
    acc = jnp.dot(a, wlT_ref[...], preferred_element_type=jnp.float32)
    acc += jnp.dot(h_ref[...], wrT_ref[...], preferred_element_type=jnp.float32)
    acc += bl_ref[...]
    if final:
        m = jnp.max(acc, axis=1, keepdims=True)
        z = acc - m
        lse = jnp.log(jnp.sum(jnp.exp(z), axis=1, keepdims=True))
        acc = z - lse
    o_ref[...] = acc


def _tc_layer(aggr, h, wlT, wrT, bl, *, final=False):
    fin = h.shape[1]
    hout = wlT.shape[1]
    return pl.pallas_call(
        functools.partial(_layer_body, final=final),
        grid=(N // ROW_BLK,),
        in_specs=[
            pl.BlockSpec((ROW_BLK, fin), lambda i: (i, 0)),
            pl.BlockSpec((ROW_BLK, fin), lambda i: (i, 0)),
            pl.BlockSpec((fin, hout), lambda i: (0, 0)),
            pl.BlockSpec((fin, hout), lambda i: (0, 0)),
            pl.BlockSpec((1, hout), lambda i: (0, 0)),
        ],
        out_specs=pl.BlockSpec((ROW_BLK, hout), lambda i: (i, 0)),
        out_shape=jax.ShapeDtypeStruct((N, hout), jnp.float32),
    )(aggr, h, wlT, wrT, bl)


def _pad2(a, r, c):
    return jnp.zeros((r, c), a.dtype).at[: a.shape[0], : a.shape[1]].set(a)


def kernel(x, edge_index, Wl1, bl1, Wr1, Wl2, bl2, Wr2, Wl3, bl3, Wr3):
    ei_flat = edge_index.reshape(2 * E)

    wlT1 = _pad2(Wl1.T, 128, 256)
    wrT1 = _pad2(Wr1.T, 128, 256)
    b1 = _pad2(bl1[None, :], 1, 256)
    wlT2 = _pad2(Wl2.T, 256, 128)
    wrT2 = _pad2(Wr2.T, 256, 128)
    b2 = _pad2(bl2[None, :], 1, 128)
    wlT3 = _pad2(Wl3.T, 128, 16)
    wrT3 = _pad2(Wr3.T, 128, 16)
    b3 = _pad2(bl3[None, :], 1, 16)

    aggr1 = _segmax_sc(x, ei_flat, 128, 8000, 128)
    h1 = _tc_layer(aggr1, x, wlT1, wrT1, b1)

    aggr2 = _segmax_sc(h1, ei_flat, 256, 4000, 32)
    h2 = _tc_layer(aggr2, h1, wlT2, wrT2, b2)

    aggr3 = _segmax_sc(h2, ei_flat, 128, 8000, 128)
    out = _tc_layer(aggr3, h2, wlT3, wrT3, b3, final=True)
    return out


# speedup vs baseline: 3.8534x; 1.0053x over previous
import dataclasses
import functools

import jax
import jax.numpy as jnp
from jax import lax
from jax.experimental import pallas as pl
from jax.experimental.pallas import tpu as pltpu
from jax.experimental.pallas import tpu_sc as plsc

N = 10000
E = 320000
NW = 32
R = 320
NPAD = NW * R
RACC = R + 4
ROW_BLK = 2000


def _segmax_sc(table, ei_flat, F, EBLK, GB):
    NCHUNK = EBLK // 16
    NBLK = E // EBLK
    GSH = GB.bit_length() - 1
    PCAP = EBLK + GB
    mesh = plsc.VectorSubcoreMesh(core_axis_name="c", subcore_axis_name="s")
    cp = pltpu.CompilerParams()
    if "needs_layout_passes" in pltpu.CompilerParams.__dataclass_fields__:
        cp = dataclasses.replace(cp, needs_layout_passes=False)

    @functools.partial(
        pl.kernel,
        out_type=jax.ShapeDtypeStruct((NPAD * F,), jnp.float32),
        mesh=mesh,
        compiler_params=cp,
        scratch_types=[
            pltpu.VMEM((RACC * F,), jnp.float32),
            pltpu.VMEM((PCAP,), jnp.int32),
            pltpu.VMEM((PCAP,), jnp.int32),
            pltpu.VMEM((EBLK,), jnp.int32),
            pltpu.VMEM((EBLK,), jnp.int32),
            pltpu.VMEM((EBLK,), jnp.int32),
            pltpu.VMEM((EBLK,), jnp.int32),
            pltpu.VMEM((GB, F), jnp.float32),
            pltpu.VMEM((GB, F), jnp.float32),
            pltpu.VMEM((32,), jnp.int32),
            pltpu.SemaphoreType.DMA,
            pltpu.SemaphoreType.DMA,
            pltpu.SemaphoreType.DMA,
            pltpu.SemaphoreType.DMA,
        ],
    )
    def k(table_hbm, ei_hbm, out_hbm, acc, psrc, pdst,
          sbuf0, dbuf0, sbuf1, dbuf1, gbuf0, gbuf1, rstage,
          semA, semB, semG0, semG1):
        wid = lax.axis_index("s") * 2 + lax.axis_index("c")
        lo = wid * R
        neg = jnp.full((16,), -jnp.inf, jnp.float32)
        iota16 = lax.iota(jnp.int32, 16)
        garb = jnp.full((16,), RACC - 1, jnp.int32)

        @pl.loop(0, RACC * F // 16)
        def _(r):
            acc[pl.ds(r * 16, 16)] = neg

        @pl.loop(0, PCAP // 16)
        def _(i):
            psrc[pl.ds(i * 16, 16)] = jnp.full((16,), wid * R, jnp.int32)
            pdst[pl.ds(i * 16, 16)] = garb

        def start_blk(b, sb, db, sem):
            pltpu.async_copy(ei_hbm.at[pl.ds(b * EBLK, EBLK)], sb, sem)
            pltpu.async_copy(ei_hbm.at[pl.ds(E + b * EBLK, EBLK)], db, sem)

        def wait_blk(sb, db, sem):
            pltpu.make_async_copy(ei_hbm.at[pl.ds(0, EBLK)], sb, sem).wait()
            pltpu.make_async_copy(ei_hbm.at[pl.ds(0, EBLK)], db, sem).wait()

        def scan(sb, db):
            def chunk(j, offv):
                dv = db[pl.ds(j * 16, 16)]
                sv = sb[pl.ds(j * 16, 16)]
                m = (dv >= lo) & (dv < lo + R)
                pos = offv + plsc.cumsum(m.astype(jnp.int32)) - 1
                pos = jnp.where(m, pos, 0)
                plsc.store_scatter(pdst, [pos], dv - lo, mask=m)
                plsc.store_scatter(psrc, [pos], sv, mask=m)
                return offv + plsc.all_reduce_population_count(m)

            return lax.fori_loop(
                0, NCHUNK, chunk, jnp.zeros((16,), jnp.int32), unroll=4)

        def start_g(b, gb, sem):
            pltpu.async_copy(table_hbm.at[psrc.at[pl.ds(b * GB, GB)]], gb, sem)

        def wait_g(gb, sem):
            pltpu.make_async_copy(
                table_hbm.at[psrc.at[pl.ds(0, GB)]], gb, sem).wait()

        def drain(offv):
            p = offv[0]
            nsub = (p + GB - 1) >> GSH

            def accum(b, gb):
                @pl.loop(0, GB // 16)
                def _(q):
                    base = b * GB + q * 16
                    dv = pdst[pl.ds(base, 16)]
                    dv = jnp.where(base + iota16 < offv, dv, garb)
                    rstage[pl.ds(16, 16)] = dv * F
                    for l in range(16):
                        roff = plsc.load_gather(
                            rstage, [jnp.full((16,), 16 + l, jnp.int32)])
                        for c in range(F // 16):
                            idx = roff + (c * 16 + iota16)
                            old = plsc.load_gather(acc, [idx])
                            new = jnp.maximum(
                                old, gb[q * 16 + l, pl.ds(c * 16, 16)])
                            plsc.store_scatter(acc, [idx], new)

            @pl.when(nsub > 0)
            def _():
                start_g(0, gbuf0, semG0)

            def pair(i, carry):
                b0 = 2 * i
                b1 = b0 + 1

                @pl.when(b1 < nsub)
                def _():
                    start_g(b1, gbuf1, semG1)

                wait_g(gbuf0, semG0)
                accum(b0, gbuf0)

                @pl.when(b0 + 2 < nsub)
                def _():
                    start_g(b0 + 2, gbuf0, semG0)

                @pl.when(b1 < nsub)
                def _():
                    wait_g(gbuf1, semG1)
                    accum(b1, gbuf1)

                return carry

            lax.fori_loop(0, (nsub + 1) >> 1, pair, jnp.int32(0))

        start_blk(0, sbuf0, dbuf0, semA)

        def blkpair(i, carry):
            b0 = 2 * i
            b1 = b0 + 1
            start_blk(b1, sbuf1, dbuf1, semB)
            wait_blk(sbuf0, dbuf0, semA)
            drain(scan(sbuf0, dbuf0))

            @pl.when(b0 + 2 < NBLK)
            def _():
                start_blk(b0 + 2, sbuf0, dbuf0, semA)

            wait_blk(sbuf1, dbuf1, semB)
            drain(scan(sbuf1, dbuf1))
            return carry

        lax.fori_loop(0, NBLK // 2, blkpair, jnp.int32(0))

        pltpu.sync_copy(
            acc.at[pl.ds(0, R * F)], out_hbm.at[pl.ds(wid * R * F, R * F)])

    return k(table, ei_flat).reshape(NPAD, F)


def _layer_body(aggr_ref, h_ref, wlT_ref, wrT_ref, bl_ref, o_ref, *, final):
    a = aggr_ref[...]
    a = jnp.where(jnp.isfinite(a), a, 0.0)
    acc = jnp.dot(a, wlT_ref[...], preferred_element_type=jnp.float32)
    acc += jnp.dot(h_ref[...], wrT_ref[...], preferred_element_type=jnp.float32)
    acc += bl_ref[...]
    if final:
        m = jnp.max(acc, axis=1, keepdims=True)
        z = acc - m
        lse = jnp.log(jnp.sum(jnp.exp(z), axis=1, keepdims=True))
        acc = z - lse
    o_ref[...] = acc


def _tc_layer(aggr, h, wlT, wrT, bl, *, final=False):
    fin = h.shape[1]
    hout = wlT.shape[1]
    return pl.pallas_call(
        functools.partial(_layer_body, final=final),
        grid=(N // ROW_BLK,),
        in_specs=[
            pl.BlockSpec((ROW_BLK, fin), lambda i: (i, 0)),
            pl.BlockSpec((ROW_BLK, fin), lambda i: (i, 0)),
            pl.BlockSpec((fin, hout), lambda i: (0, 0)),
            pl.BlockSpec((fin, hout), lambda i: (0, 0)),
            pl.BlockSpec((1, hout), lambda i: (0, 0)),
        ],
        out_specs=pl.BlockSpec((ROW_BLK, hout), lambda i: (i, 0)),
        out_shape=jax.ShapeDtypeStruct((N, hout), jnp.float32),
    )(aggr, h, wlT, wrT, bl)


def _pad2(a, r, c):
    return jnp.zeros((r, c), a.dtype).at[: a.shape[0], : a.shape[1]].set(a)


def kernel(x, edge_index, Wl1, bl1, Wr1, Wl2, bl2, Wr2, Wl3, bl3, Wr3):
    ei_flat = edge_index.reshape(2 * E)

    wlT1 = _pad2(Wl1.T, 128, 256)
    wrT1 = _pad2(Wr1.T, 128, 256)
    b1 = _pad2(bl1[None, :], 1, 256)
    wlT2 = _pad2(Wl2.T, 256, 128)
    wrT2 = _pad2(Wr2.T, 256, 128)
    b2 = _pad2(bl2[None, :], 1, 128)
    wlT3 = _pad2(Wl3.T, 128, 16)
    wrT3 = _pad2(Wr3.T, 128, 16)
    b3 = _pad2(bl3[None, :], 1, 16)

    aggr1 = _segmax_sc(x, ei_flat, 128, 8000, 128)
    h1 = _tc_layer(aggr1, x, wlT1, wrT1, b1)

    aggr2 = _segmax_sc(h1, ei_flat, 256, 4000, 32)
    h2 = _tc_layer(aggr2, h1, wlT2, wrT2, b2)

    aggr3 = _segmax_sc(h2, ei_flat, 128, 8000, 128)
    out = _tc_layer(aggr3, h2, wlT3, wrT3, b3, final=True)
    return out
